# baseline (speedup 1.0000x reference)
import functools

import jax
import jax.numpy as jnp
from jax import lax
from jax.experimental import pallas as pl
from jax.experimental.pallas import tpu as pltpu
from jax.experimental.pallas import tpu_sc as plsc

NUM_CLASSES = 1000
CAP = 32
DIM = 1024
BATCH = 4096
GET = 16
L = 16
CHUNKS = BATCH // L
OUT_ROWS = GET * CAP + 2 * GET


def _sc_body(x_hbm, mem_hbm, cls_hbm, coll_hbm, out_hbm,
             cls_v, coll_v, slot_v, midx_v, dx_v, dm_v, xrows_v, mrows_v,
             sem_a, sem_b):
    wid = lax.axis_index("s") * 2 + lax.axis_index("c")

    @pl.when(wid < GET)
    def _():
        pltpu.sync_copy(cls_hbm, cls_v)
        pltpu.sync_copy(coll_hbm, coll_v)
        widv = jnp.full((L,), wid, jnp.int32)
        ck = plsc.load_gather(coll_v, [widv])
        zeros = jnp.zeros((L,), jnp.int32)
        slot_v[pl.ds(0, L)] = zeros
        slot_v[pl.ds(L, L)] = zeros
        lanes = lax.iota(jnp.int32, L)

        def step(j, offv):
            v = cls_v[pl.ds(j * L, L)]
            m = v == ck

            def hit(o):
                mi = m.astype(jnp.int32)
                incl = plsc.cumsum(mi)
                ranks = o + incl - mi
                plsc.store_scatter(slot_v, [ranks], lanes + j * L,
                                   mask=m & (ranks < CAP))
                return o + plsc.all_reduce_population_count(m)

            return lax.cond(jnp.any(m), hit, lambda o: o, offv)

        countv = lax.fori_loop(0, CHUNKS, step, zeros)

        base = wid * CAP
        xtrash = GET * CAP + wid
        mtrash = GET * CAP + GET + wid
        for h in range(CAP // L):
            s_v = lanes + h * L
            use_x = s_v < countv
            dx = jnp.where(use_x, base + s_v, xtrash)
            dm = jnp.where(use_x, mtrash, base + s_v)
            mid = ck * CAP + jnp.clip(s_v - countv, 0, CAP - 1)
            dx_v[pl.ds(h * L, L)] = dx
            dm_v[pl.ds(h * L, L)] = dm
            midx_v[pl.ds(h * L, L)] = mid

        g1 = pltpu.async_copy(x_hbm.at[slot_v], xrows_v, sem_a)
        g2 = pltpu.async_copy(mem_hbm.at[midx_v], mrows_v, sem_b)
        g1.wait()
        s1 = pltpu.async_copy(xrows_v, out_hbm.at[dx_v], sem_a)
        g2.wait()
        s2 = pltpu.async_copy(mrows_v, out_hbm.at[dm_v], sem_b)
        s1.wait()
        s2.wait()


_sc_call = functools.partial(
    pl.kernel,
    out_type=jax.ShapeDtypeStruct((OUT_ROWS, DIM), jnp.float32),
    mesh=plsc.VectorSubcoreMesh(core_axis_name="c", subcore_axis_name="s"),
    compiler_params=pltpu.CompilerParams(needs_layout_passes=False),
    scratch_types=[
        pltpu.VMEM((BATCH,), jnp.int32),
        pltpu.VMEM((L,), jnp.int32),
        pltpu.VMEM((CAP,), jnp.int32),
        pltpu.VMEM((CAP,), jnp.int32),
        pltpu.VMEM((CAP,), jnp.int32),
        pltpu.VMEM((CAP,), jnp.int32),
        pltpu.VMEM((CAP, DIM), jnp.float32),
        pltpu.VMEM((CAP, DIM), jnp.float32),
        pltpu.SemaphoreType.DMA,
        pltpu.SemaphoreType.DMA,
    ],
)(_sc_body)


def kernel(x, classes, get_cls, memory):
    num_classes, cap, dim = memory.shape
    collected = jax.random.randint(jax.random.key(1), (GET,), 0, num_classes)
    memflat = memory.reshape(num_classes * cap, dim)
    out = _sc_call(x, memflat, classes.astype(jnp.int32),
                   collected.astype(jnp.int32))
    return out[:GET * CAP].reshape(GET, cap, dim)


# speedup vs baseline: 1.0562x; 1.0562x over previous
import functools

import jax
import jax.numpy as jnp
from jax import lax
from jax.experimental import pallas as pl
from jax.experimental.pallas import tpu as pltpu
from jax.experimental.pallas import tpu_sc as plsc

NUM_CLASSES = 1000
CAP = 32
DIM = 1024
BATCH = 4096
GET = 16
L = 16
NW = 32
CHUNKS = BATCH // L
OUT_ROWS = GET * CAP + 2 * NW


def _sc_body(x_hbm, mem_hbm, cls_hbm, coll_hbm, out_hbm,
             cls_v, coll_v, slot_v, xidx_v, midx_v, dx_v, dm_v,
             xrows_v, mrows_v, sem_a, sem_b):
    wid = lax.axis_index("s") * 2 + lax.axis_index("c")
    k = wid & (GET - 1)
    h = wid >> 4

    pltpu.sync_copy(cls_hbm, cls_v)
    pltpu.sync_copy(coll_hbm, coll_v)
    kv = jnp.full((L,), k, jnp.int32)
    ck = plsc.load_gather(coll_v, [kv])
    zeros = jnp.zeros((L,), jnp.int32)
    slot_v[pl.ds(0, L)] = zeros
    slot_v[pl.ds(L, L)] = zeros
    lanes = lax.iota(jnp.int32, L)

    def step(j, offv):
        v = cls_v[pl.ds(j * L, L)]
        m = v == ck
        mi = m.astype(jnp.int32)
        incl = plsc.cumsum(mi)
        ranks = offv + incl - mi
        plsc.store_scatter(slot_v, [ranks], lanes + j * L,
                           mask=m & (ranks < CAP))
        return offv + plsc.all_reduce_population_count(m)

    countv = lax.fori_loop(0, CHUNKS, step, zeros)

    s_v = lanes + h * L
    base = k * CAP
    use_x = s_v < countv
    dx = jnp.where(use_x, base + s_v, GET * CAP + wid)
    dm = jnp.where(use_x, GET * CAP + NW + wid, base + s_v)
    mid = ck * CAP + jnp.clip(s_v - countv, 0, CAP - 1)
    xidx_v[pl.ds(0, L)] = slot_v[pl.ds(h * L, L)]
    dx_v[pl.ds(0, L)] = dx
    dm_v[pl.ds(0, L)] = dm
    midx_v[pl.ds(0, L)] = mid

    g1 = pltpu.async_copy(x_hbm.at[xidx_v], xrows_v, sem_a)
    g2 = pltpu.async_copy(mem_hbm.at[midx_v], mrows_v, sem_b)
    g1.wait()
    s1 = pltpu.async_copy(xrows_v, out_hbm.at[dx_v], sem_a)
    g2.wait()
    s2 = pltpu.async_copy(mrows_v, out_hbm.at[dm_v], sem_b)
    s1.wait()
    s2.wait()


_sc_call = functools.partial(
    pl.kernel,
    out_type=jax.ShapeDtypeStruct((OUT_ROWS, DIM), jnp.float32),
    mesh=plsc.VectorSubcoreMesh(core_axis_name="c", subcore_axis_name="s"),
    compiler_params=pltpu.CompilerParams(needs_layout_passes=False),
    scratch_types=[
        pltpu.VMEM((BATCH,), jnp.int32),
        pltpu.VMEM((L,), jnp.int32),
        pltpu.VMEM((CAP,), jnp.int32),
        pltpu.VMEM((L,), jnp.int32),
        pltpu.VMEM((L,), jnp.int32),
        pltpu.VMEM((L,), jnp.int32),
        pltpu.VMEM((L,), jnp.int32),
        pltpu.VMEM((L, DIM), jnp.float32),
        pltpu.VMEM((L, DIM), jnp.float32),
        pltpu.SemaphoreType.DMA,
        pltpu.SemaphoreType.DMA,
    ],
)(_sc_body)


def kernel(x, classes, get_cls, memory):
    num_classes, cap, dim = memory.shape
    collected = jax.random.randint(jax.random.key(1), (GET,), 0, num_classes)
    memflat = memory.reshape(num_classes * cap, dim)
    out = _sc_call(x, memflat, classes.astype(jnp.int32),
                   collected.astype(jnp.int32))
    return out[:GET * CAP].reshape(GET, cap, dim)


# speedup vs baseline: 1.1837x; 1.1207x over previous
import functools

import jax
import jax.numpy as jnp
import numpy as np
from jax import lax
from jax.experimental import pallas as pl
from jax.experimental.pallas import tpu as pltpu
from jax.experimental.pallas import tpu_sc as plsc

NUM_CLASSES = 1000
CAP = 32
DIM = 1024
BATCH = 4096
GET = 16
L = 16
NW = 32
CHUNKS = BATCH // L
OUT_ROWS = GET * CAP + 2 * NW

_COLLECTED = np.asarray(
    [996, 927, 40, 353, 768, 684, 438, 381, 506, 946,
     408, 33, 874, 930, 398, 226], dtype=np.int32)


def _sc_body(x_hbm, mem_hbm, cls_hbm, out_hbm,
             cls_v, slot_v, xidx_v, midx_v, dst_v, rows_v, sem_a, sem_b):
    wid = lax.axis_index("s") * 2 + lax.axis_index("c")
    k = wid & (GET - 1)
    h = wid >> 4

    gc = pltpu.async_copy(cls_hbm, cls_v, sem_a)

    ck_s = jnp.int32(int(_COLLECTED[0]))
    for i in range(1, GET):
        ck_s = jnp.where(k == i, jnp.int32(int(_COLLECTED[i])), ck_s)
    ck = jnp.full((L,), ck_s, jnp.int32)
    lanes = lax.iota(jnp.int32, L)
    j_v = lanes + h * L
    midx_v[pl.ds(0, L)] = ck * CAP + j_v
    gm = pltpu.async_copy(mem_hbm.at[midx_v], rows_v.at[pl.ds(L, L)], sem_b)

    zeros = jnp.zeros((L,), jnp.int32)
    slot_v[pl.ds(0, L)] = zeros
    slot_v[pl.ds(L, L)] = zeros

    gc.wait()

    def step(j, offv):
        v = cls_v[pl.ds(j * L, L)]
        m = v == ck
        mi = m.astype(jnp.int32)
        incl = plsc.cumsum(mi)
        ranks = offv + incl - mi
        plsc.store_scatter(slot_v, [ranks], lanes + j * L,
                           mask=m & (ranks < CAP))
        return offv + plsc.all_reduce_population_count(m)

    countv = lax.fori_loop(0, CHUNKS, step, zeros)

    s_v = j_v
    base = k * CAP
    dx = jnp.where(s_v < countv, base + s_v, GET * CAP + wid)
    dm = jnp.where(j_v + countv < CAP, base + j_v + countv,
                   GET * CAP + NW + wid)
    xidx_v[pl.ds(0, L)] = slot_v[pl.ds(h * L, L)]
    dst_v[pl.ds(0, L)] = dx
    dst_v[pl.ds(L, L)] = dm

    gx = pltpu.async_copy(x_hbm.at[xidx_v], rows_v.at[pl.ds(0, L)], sem_a)
    gx.wait()
    gm.wait()
    sc = pltpu.async_copy(rows_v, out_hbm.at[dst_v], sem_a)
    sc.wait()


_sc_call = functools.partial(
    pl.kernel,
    out_type=jax.ShapeDtypeStruct((OUT_ROWS, DIM), jnp.float32),
    mesh=plsc.VectorSubcoreMesh(core_axis_name="c", subcore_axis_name="s"),
    compiler_params=pltpu.CompilerParams(needs_layout_passes=False),
    scratch_types=[
        pltpu.VMEM((BATCH,), jnp.int32),
        pltpu.VMEM((CAP,), jnp.int32),
        pltpu.VMEM((L,), jnp.int32),
        pltpu.VMEM((L,), jnp.int32),
        pltpu.VMEM((CAP,), jnp.int32),
        pltpu.VMEM((CAP, DIM), jnp.float32),
        pltpu.SemaphoreType.DMA,
        pltpu.SemaphoreType.DMA,
    ],
)(_sc_body)


def kernel(x, classes, get_cls, memory):
    num_classes, cap, dim = memory.shape
    memflat = memory.reshape(num_classes * cap, dim)
    out = _sc_call(x, memflat, classes.astype(jnp.int32))
    return out[:GET * CAP].reshape(GET, cap, dim)
